# baseline (speedup 1.0000x reference)
import functools

import jax
import jax.numpy as jnp
from jax import lax
from jax.experimental import pallas as pl
from jax.experimental.pallas import tpu as pltpu
from jax.experimental.pallas import tpu_sc as plsc

N = 10000
D = 128
E = 320000
NC = 2
NS = 16
NW = NC * NS
CHUNK = 128
K = 80
E_PAD = NW * K * CHUNK
ROWS_PER_TILE = 624
ZCHUNK = 104
NZ = 6
TAIL_OFF = NS * ROWS_PER_TILE
TAIL = N - TAIL_OFF


def _mm_body(x_ref, w_ref, o_ref):
    o_ref[...] = jnp.dot(x_ref[...], w_ref[...],
                         preferred_element_type=jnp.float32)


def _matmul(x, W):
    return pl.pallas_call(
        _mm_body,
        grid=(10,),
        in_specs=[
            pl.BlockSpec((N // 10, D), lambda i: (i, 0)),
            pl.BlockSpec((D, D), lambda i: (0, 0)),
        ],
        out_specs=pl.BlockSpec((N // 10, D), lambda i: (i, 0)),
        out_shape=jax.ShapeDtypeStruct((N, D), jnp.float32),
    )(x, W)


def _sum_body(p_ref, o_ref):
    o_ref[...] = p_ref[0] + p_ref[1]


def _sum_partials(partials):
    return pl.pallas_call(
        _sum_body,
        grid=(10,),
        in_specs=[pl.BlockSpec((NC, N // 10, D), lambda i: (0, i, 0))],
        out_specs=pl.BlockSpec((N // 10, D), lambda i: (i, 0)),
        out_shape=jax.ShapeDtypeStruct((N, D), jnp.float32),
    )(partials)


def _spmm_body(support_hbm, src_hbm, dst_hbm, ev_hbm, out_hbm,
               src_v, dst_v, ev_v, rows_v, acc,
               gsem0, gsem1, msem0, msem1, csem0, csem1):
    c = lax.axis_index("c")
    s = lax.axis_index("s")
    wid = s * NC + c
    rows0, rows1 = rows_v.at[0], rows_v.at[1]

    def _zrow(i, carry):
        for g in range(8):
            rows_v[0, i, pl.ds(g * 16, 16)] = jnp.zeros((16,), jnp.float32)
        return carry
    with jax.named_scope("sc_zero"):
        lax.fori_loop(0, CHUNK, _zrow, 0)

    with jax.named_scope("sc_init"):
        pltpu.sync_copy(src_hbm.at[wid], src_v)
        for kz in range(NZ):
            off = s * ROWS_PER_TILE + kz * ZCHUNK
            pltpu.sync_copy(rows0.at[pl.ds(0, ZCHUNK)],
                            acc.at[pl.ds(off, ZCHUNK)])

        @pl.when(s == 0)
        def _zero_tail():
            pltpu.sync_copy(rows0.at[pl.ds(0, TAIL)],
                            acc.at[pl.ds(TAIL_OFF, TAIL)])
        plsc.subcore_barrier()

    def _start(j, b, rows_b, gsem, msem):
        pltpu.async_copy(support_hbm.at[src_v.at[j]], rows_b, gsem)
        pltpu.async_copy(dst_hbm.at[wid, j], dst_v.at[b], msem)
        pltpu.async_copy(ev_hbm.at[wid, j], ev_v.at[b], msem)

    def _wait(j, b, rows_b, gsem, msem):
        pltpu.make_async_copy(support_hbm.at[src_v.at[j]], rows_b, gsem).wait()
        pltpu.make_async_copy(dst_hbm.at[wid, j], dst_v.at[b], msem).wait()
        pltpu.make_async_copy(ev_hbm.at[wid, j], ev_v.at[b], msem).wait()

    def _scale_scatter(b, rows_b):
        ev_b = ev_v.at[b]

        @plsc.parallel_loop(0, CHUNK // 16, unroll=2)
        def _scale(g16):
            vals16 = ev_b[pl.ds(g16 * 16, 16)]
            for l in range(16):
                e = g16 * 16 + l
                bc = jnp.broadcast_to(vals16[l], (16,))
                for g in range(8):
                    sl = pl.ds(g * 16, 16)
                    rows_b[e, sl] = rows_b[e, sl] * bc
        pltpu.async_copy(rows_b, acc.at[dst_v.at[b]], (csem0 if b == 0 else csem1), add=True)

    def _scatter_wait(b, rows_b):
        pltpu.make_async_copy(rows_b, acc.at[dst_v.at[b]], (csem0 if b == 0 else csem1)).wait()

    with jax.named_scope("sc_prime"):
        _start(0, 0, rows0, gsem0, msem0)

    def _pair(i, carry):
        jj = 2 * i

        @pl.when(i > 0)
        def _drain_prev_scatter1():
            _scatter_wait(1, rows1)
        _start(jj + 1, 1, rows1, gsem1, msem1)
        _wait(jj, 0, rows0, gsem0, msem0)
        _scale_scatter(0, rows0)
        _wait(jj + 1, 1, rows1, gsem1, msem1)
        _scale_scatter(1, rows1)

        @pl.when(i + 1 < K // 2)
        def _prefetch_next():
            _scatter_wait(0, rows0)
            _start(jj + 2, 0, rows0, gsem0, msem0)
        return carry
    with jax.named_scope("sc_main"):
        lax.fori_loop(0, K // 2, _pair, 0)
        _scatter_wait(0, rows0)
        _scatter_wait(1, rows1)

    with jax.named_scope("sc_bar2"):
        plsc.subcore_barrier()
    with jax.named_scope("sc_wb"):
        for kz in range(NZ):
            off = s * ROWS_PER_TILE + kz * ZCHUNK
            pltpu.sync_copy(acc.at[pl.ds(off, ZCHUNK)],
                            out_hbm.at[c, pl.ds(off, ZCHUNK)])

    @pl.when(s == 0)
    def _write_tail():
        pltpu.sync_copy(acc.at[pl.ds(TAIL_OFF, TAIL)],
                        out_hbm.at[c, pl.ds(TAIL_OFF, TAIL)])


_spmm = pl.kernel(
    _spmm_body,
    out_type=jax.ShapeDtypeStruct((NC, N, D), jnp.float32),
    mesh=plsc.VectorSubcoreMesh(core_axis_name="c", subcore_axis_name="s"),
    scratch_types=[
        pltpu.VMEM((K, CHUNK), jnp.int32),
        pltpu.VMEM((2, CHUNK), jnp.int32),
        pltpu.VMEM((2, CHUNK), jnp.float32),
        pltpu.VMEM((2, CHUNK, D), jnp.float32),
        pltpu.VMEM_SHARED((N, D), jnp.float32),
        pltpu.SemaphoreType.DMA,
        pltpu.SemaphoreType.DMA,
        pltpu.SemaphoreType.DMA,
        pltpu.SemaphoreType.DMA,
        pltpu.SemaphoreType.DMA,
        pltpu.SemaphoreType.DMA,
    ],
)


@jax.jit
def kernel(x, edge_index, edge_values, W):
    support = _matmul(x, W)
    dst = edge_index[0]
    src = edge_index[1]
    pad = E_PAD - E
    zi = (jnp.arange(pad, dtype=jnp.int32) * 13) % N
    src_p = jnp.concatenate([src, zi]).reshape(NW, K, CHUNK)
    dst_p = jnp.concatenate([dst, zi]).reshape(NW, K, CHUNK)
    ev_p = jnp.concatenate([edge_values, jnp.zeros((pad,), jnp.float32)]
                           ).reshape(NW, K, CHUNK)
    partials = _spmm(support, src_p, dst_p, ev_p)
    return _sum_partials(partials)


# speedup vs baseline: 1.1837x; 1.1837x over previous
import functools

import jax
import jax.numpy as jnp
from jax import lax
from jax.experimental import pallas as pl
from jax.experimental.pallas import tpu as pltpu
from jax.experimental.pallas import tpu_sc as plsc

N = 10000
D = 128
E = 320000
NC = 2
NS = 16
NW = NC * NS
CHUNK = 128
K = 80
E_PAD = NW * K * CHUNK
ROWS_PER_TILE = 624
ZCHUNK = 104
NZ = 6
TAIL_OFF = NS * ROWS_PER_TILE
TAIL = N - TAIL_OFF


NCHUNKS = E_PAD // CHUNK
CB = NCHUNKS // 10


def _mm_body(x_ref, w_ref, ei_ref, ev_ref,
             sup_ref, src_ref, dst_ref, evp_ref):
    sup_ref[...] = jnp.dot(x_ref[...], w_ref[...],
                           preferred_element_type=jnp.float32)
    i = pl.program_id(0)
    r = lax.broadcasted_iota(jnp.int32, (CB, CHUNK), 0)
    cl = lax.broadcasted_iota(jnp.int32, (CB, CHUNK), 1)
    pos = (i * CB + r) * CHUNK + cl
    valid = pos < E
    spread = (pos * 13) % N
    src_ref[...] = jnp.where(valid, ei_ref[1], spread)
    dst_ref[...] = jnp.where(valid, ei_ref[0], spread)
    evp_ref[...] = jnp.where(valid, ev_ref[...], 0.0)


def _matmul_prep(x, W, edge_index, edge_values):
    ei3 = edge_index.reshape(2, E // CHUNK, CHUNK)
    ev2 = edge_values.reshape(E // CHUNK, CHUNK)
    return pl.pallas_call(
        _mm_body,
        grid=(10,),
        in_specs=[
            pl.BlockSpec((N // 10, D), lambda i: (i, 0)),
            pl.BlockSpec((D, D), lambda i: (0, 0)),
            pl.BlockSpec((2, CB, CHUNK), lambda i: (0, i, 0)),
            pl.BlockSpec((CB, CHUNK), lambda i: (i, 0)),
        ],
        out_specs=[
            pl.BlockSpec((N // 10, D), lambda i: (i, 0)),
            pl.BlockSpec((CB, CHUNK), lambda i: (i, 0)),
            pl.BlockSpec((CB, CHUNK), lambda i: (i, 0)),
            pl.BlockSpec((CB, CHUNK), lambda i: (i, 0)),
        ],
        out_shape=[
            jax.ShapeDtypeStruct((N, D), jnp.float32),
            jax.ShapeDtypeStruct((NCHUNKS, CHUNK), jnp.int32),
            jax.ShapeDtypeStruct((NCHUNKS, CHUNK), jnp.int32),
            jax.ShapeDtypeStruct((NCHUNKS, CHUNK), jnp.float32),
        ],
    )(x, W, ei3, ev2)


def _sum_body(p_ref, o_ref):
    o_ref[...] = p_ref[0] + p_ref[1]


def _sum_partials(partials):
    return pl.pallas_call(
        _sum_body,
        grid=(10,),
        in_specs=[pl.BlockSpec((NC, N // 10, D), lambda i: (0, i, 0))],
        out_specs=pl.BlockSpec((N // 10, D), lambda i: (i, 0)),
        out_shape=jax.ShapeDtypeStruct((N, D), jnp.float32),
    )(partials)


def _spmm_body(support_hbm, src_hbm, dst_hbm, ev_hbm, out_hbm,
               src_v, dst_v, ev_v, rows_v, acc,
               gsem0, gsem1, msem0, msem1):
    c = lax.axis_index("c")
    s = lax.axis_index("s")
    wid = s * NC + c
    rows0, rows1 = rows_v.at[0], rows_v.at[1]

    def _zrow(i, carry):
        for g in range(8):
            rows_v[0, i, pl.ds(g * 16, 16)] = jnp.zeros((16,), jnp.float32)
        return carry
    with jax.named_scope("sc_zero"):
        lax.fori_loop(0, CHUNK, _zrow, 0)

    with jax.named_scope("sc_init"):
        pltpu.sync_copy(src_hbm.at[pl.ds(wid * K, K)], src_v)
        for kz in range(NZ):
            off = s * ROWS_PER_TILE + kz * ZCHUNK
            pltpu.sync_copy(rows0.at[pl.ds(0, ZCHUNK)],
                            acc.at[pl.ds(off, ZCHUNK)])

        @pl.when(s == 0)
        def _zero_tail():
            pltpu.sync_copy(rows0.at[pl.ds(0, TAIL)],
                            acc.at[pl.ds(TAIL_OFF, TAIL)])
        plsc.subcore_barrier()

    def _start(j, b, rows_b, gsem, msem):
        pltpu.async_copy(support_hbm.at[src_v.at[j]], rows_b, gsem)
        pltpu.async_copy(dst_hbm.at[wid * K + j], dst_v.at[b], msem)
        pltpu.async_copy(ev_hbm.at[wid * K + j], ev_v.at[b], msem)

    def _wait(j, b, rows_b, gsem, msem):
        pltpu.make_async_copy(support_hbm.at[src_v.at[j]], rows_b, gsem).wait()
        pltpu.make_async_copy(dst_hbm.at[wid * K + j], dst_v.at[b], msem).wait()
        pltpu.make_async_copy(ev_hbm.at[wid * K + j], ev_v.at[b], msem).wait()

    def _scale_scatter(b, rows_b):
        ev_b = ev_v.at[b]

        @plsc.parallel_loop(0, CHUNK // 16, unroll=2)
        def _scale(g16):
            vals16 = ev_b[pl.ds(g16 * 16, 16)]
            for l in range(16):
                e = g16 * 16 + l
                bc = jnp.broadcast_to(vals16[l], (16,))
                for g in range(8):
                    sl = pl.ds(g * 16, 16)
                    rows_b[e, sl] = rows_b[e, sl] * bc
        pltpu.sync_copy(rows_b, acc.at[dst_v.at[b]], add=True)

    with jax.named_scope("sc_prime"):
        _start(0, 0, rows0, gsem0, msem0)

    def _pair(i, carry):
        jj = 2 * i
        _start(jj + 1, 1, rows1, gsem1, msem1)
        _wait(jj, 0, rows0, gsem0, msem0)
        _scale_scatter(0, rows0)

        @pl.when(i + 1 < K // 2)
        def _prefetch_next():
            _start(jj + 2, 0, rows0, gsem0, msem0)
        _wait(jj + 1, 1, rows1, gsem1, msem1)
        _scale_scatter(1, rows1)
        return carry
    with jax.named_scope("sc_main"):
        lax.fori_loop(0, K // 2, _pair, 0)

    with jax.named_scope("sc_bar2"):
        plsc.subcore_barrier()
    with jax.named_scope("sc_wb"):
        for kz in range(NZ):
            off = s * ROWS_PER_TILE + kz * ZCHUNK
            pltpu.sync_copy(acc.at[pl.ds(off, ZCHUNK)],
                            out_hbm.at[c, pl.ds(off, ZCHUNK)])

    @pl.when(s == 0)
    def _write_tail():
        pltpu.sync_copy(acc.at[pl.ds(TAIL_OFF, TAIL)],
                        out_hbm.at[c, pl.ds(TAIL_OFF, TAIL)])


_spmm = pl.kernel(
    _spmm_body,
    out_type=jax.ShapeDtypeStruct((NC, N, D), jnp.float32),
    mesh=plsc.VectorSubcoreMesh(core_axis_name="c", subcore_axis_name="s"),
    scratch_types=[
        pltpu.VMEM((K, CHUNK), jnp.int32),
        pltpu.VMEM((2, CHUNK), jnp.int32),
        pltpu.VMEM((2, CHUNK), jnp.float32),
        pltpu.VMEM((2, CHUNK, D), jnp.float32),
        pltpu.VMEM_SHARED((N, D), jnp.float32),
        pltpu.SemaphoreType.DMA,
        pltpu.SemaphoreType.DMA,
        pltpu.SemaphoreType.DMA,
        pltpu.SemaphoreType.DMA,
    ],
)


@jax.jit
def kernel(x, edge_index, edge_values, W):
    support, src_p, dst_p, ev_p = _matmul_prep(x, W, edge_index, edge_values)
    partials = _spmm(support, src_p, dst_p, ev_p)
    return _sum_partials(partials)


# speedup vs baseline: 1.2157x; 1.0270x over previous
import functools

import jax
import jax.numpy as jnp
from jax import lax
from jax.experimental import pallas as pl
from jax.experimental.pallas import tpu as pltpu
from jax.experimental.pallas import tpu_sc as plsc

N = 10000
D = 128
E = 320000
NC = 2
NS = 16
NW = NC * NS
CHUNK = 128
K = 80
E_PAD = NW * K * CHUNK
ROWS_PER_TILE = 624
ZCHUNK = 104
NZ = 6
TAIL_OFF = NS * ROWS_PER_TILE
TAIL = N - TAIL_OFF


NCHUNKS = E_PAD // CHUNK
CB = NCHUNKS // 10


def _mm_body(x_ref, w_ref, ei_ref, ev_ref,
             sup_ref, src_ref, dst_ref, evp_ref):
    sup_ref[...] = jnp.dot(x_ref[...], w_ref[...],
                           preferred_element_type=jnp.float32)
    i = pl.program_id(0)
    r = lax.broadcasted_iota(jnp.int32, (CB, CHUNK), 0)
    cl = lax.broadcasted_iota(jnp.int32, (CB, CHUNK), 1)
    pos = (i * CB + r) * CHUNK + cl
    valid = pos < E
    spread = (pos * 13) % N
    src_ref[...] = jnp.where(valid, ei_ref[1].reshape(CB, CHUNK), spread)
    dst_ref[...] = jnp.where(valid, ei_ref[0].reshape(CB, CHUNK), spread)
    evp_ref[...] = jnp.where(valid, ev_ref[...].reshape(CB, CHUNK), 0.0)


def _matmul_prep(x, W, edge_index, edge_values):
    return pl.pallas_call(
        _mm_body,
        grid=(10,),
        in_specs=[
            pl.BlockSpec((N // 10, D), lambda i: (i, 0)),
            pl.BlockSpec((D, D), lambda i: (0, 0)),
            pl.BlockSpec((2, CB * CHUNK), lambda i: (0, i)),
            pl.BlockSpec((CB * CHUNK,), lambda i: (i,)),
        ],
        out_specs=[
            pl.BlockSpec((N // 10, D), lambda i: (i, 0)),
            pl.BlockSpec((CB, CHUNK), lambda i: (i, 0)),
            pl.BlockSpec((CB, CHUNK), lambda i: (i, 0)),
            pl.BlockSpec((CB, CHUNK), lambda i: (i, 0)),
        ],
        out_shape=[
            jax.ShapeDtypeStruct((N, D), jnp.float32),
            jax.ShapeDtypeStruct((NCHUNKS, CHUNK), jnp.int32),
            jax.ShapeDtypeStruct((NCHUNKS, CHUNK), jnp.int32),
            jax.ShapeDtypeStruct((NCHUNKS, CHUNK), jnp.float32),
        ],
    )(x, W, edge_index, edge_values)


def _sum_body(p_ref, o_ref):
    o_ref[...] = p_ref[0] + p_ref[1]


def _sum_partials(partials):
    return pl.pallas_call(
        _sum_body,
        grid=(10,),
        in_specs=[pl.BlockSpec((NC, N // 10, D), lambda i: (0, i, 0))],
        out_specs=pl.BlockSpec((N // 10, D), lambda i: (i, 0)),
        out_shape=jax.ShapeDtypeStruct((N, D), jnp.float32),
    )(partials)


def _spmm_body(support_hbm, src_hbm, dst_hbm, ev_hbm, out_hbm,
               src_v, dst_v, ev_v, rows_v, acc,
               gsem0, gsem1, msem0, msem1):
    c = lax.axis_index("c")
    s = lax.axis_index("s")
    wid = s * NC + c
    rows0, rows1 = rows_v.at[0], rows_v.at[1]

    def _zrow(i, carry):
        for g in range(8):
            rows_v[0, i, pl.ds(g * 16, 16)] = jnp.zeros((16,), jnp.float32)
        return carry
    with jax.named_scope("sc_zero"):
        lax.fori_loop(0, CHUNK, _zrow, 0)

    with jax.named_scope("sc_init"):
        pltpu.sync_copy(src_hbm.at[pl.ds(wid * K, K)], src_v)
        for kz in range(NZ):
            off = s * ROWS_PER_TILE + kz * ZCHUNK
            pltpu.sync_copy(rows0.at[pl.ds(0, ZCHUNK)],
                            acc.at[pl.ds(off, ZCHUNK)])

        @pl.when(s == 0)
        def _zero_tail():
            pltpu.sync_copy(rows0.at[pl.ds(0, TAIL)],
                            acc.at[pl.ds(TAIL_OFF, TAIL)])
        plsc.subcore_barrier()

    def _start(j, b, rows_b, gsem, msem):
        pltpu.async_copy(support_hbm.at[src_v.at[j]], rows_b, gsem)
        pltpu.async_copy(dst_hbm.at[wid * K + j], dst_v.at[b], msem)
        pltpu.async_copy(ev_hbm.at[wid * K + j], ev_v.at[b], msem)

    def _wait(j, b, rows_b, gsem, msem):
        pltpu.make_async_copy(support_hbm.at[src_v.at[j]], rows_b, gsem).wait()
        pltpu.make_async_copy(dst_hbm.at[wid * K + j], dst_v.at[b], msem).wait()
        pltpu.make_async_copy(ev_hbm.at[wid * K + j], ev_v.at[b], msem).wait()

    def _scale_scatter(b, rows_b):
        ev_b = ev_v.at[b]

        @plsc.parallel_loop(0, CHUNK // 16, unroll=2)
        def _scale(g16):
            vals16 = ev_b[pl.ds(g16 * 16, 16)]
            for l in range(16):
                e = g16 * 16 + l
                bc = jnp.broadcast_to(vals16[l], (16,))
                for g in range(8):
                    sl = pl.ds(g * 16, 16)
                    rows_b[e, sl] = rows_b[e, sl] * bc
        pltpu.sync_copy(rows_b, acc.at[dst_v.at[b]], add=True)

    with jax.named_scope("sc_prime"):
        _start(0, 0, rows0, gsem0, msem0)

    def _pair(i, carry):
        jj = 2 * i
        _start(jj + 1, 1, rows1, gsem1, msem1)
        _wait(jj, 0, rows0, gsem0, msem0)
        _scale_scatter(0, rows0)

        @pl.when(i + 1 < K // 2)
        def _prefetch_next():
            _start(jj + 2, 0, rows0, gsem0, msem0)
        _wait(jj + 1, 1, rows1, gsem1, msem1)
        _scale_scatter(1, rows1)
        return carry
    with jax.named_scope("sc_main"):
        lax.fori_loop(0, K // 2, _pair, 0)

    with jax.named_scope("sc_bar2"):
        plsc.subcore_barrier()
    with jax.named_scope("sc_wb"):
        for kz in range(NZ):
            off = s * ROWS_PER_TILE + kz * ZCHUNK
            pltpu.sync_copy(acc.at[pl.ds(off, ZCHUNK)],
                            out_hbm.at[c, pl.ds(off, ZCHUNK)])

    @pl.when(s == 0)
    def _write_tail():
        pltpu.sync_copy(acc.at[pl.ds(TAIL_OFF, TAIL)],
                        out_hbm.at[c, pl.ds(TAIL_OFF, TAIL)])


_spmm = pl.kernel(
    _spmm_body,
    out_type=jax.ShapeDtypeStruct((NC, N, D), jnp.float32),
    mesh=plsc.VectorSubcoreMesh(core_axis_name="c", subcore_axis_name="s"),
    scratch_types=[
        pltpu.VMEM((K, CHUNK), jnp.int32),
        pltpu.VMEM((2, CHUNK), jnp.int32),
        pltpu.VMEM((2, CHUNK), jnp.float32),
        pltpu.VMEM((2, CHUNK, D), jnp.float32),
        pltpu.VMEM_SHARED((N, D), jnp.float32),
        pltpu.SemaphoreType.DMA,
        pltpu.SemaphoreType.DMA,
        pltpu.SemaphoreType.DMA,
        pltpu.SemaphoreType.DMA,
    ],
)


@jax.jit
def kernel(x, edge_index, edge_values, W):
    support, src_p, dst_p, ev_p = _matmul_prep(x, W, edge_index, edge_values)
    partials = _spmm(support, src_p, dst_p, ev_p)
    return _sum_partials(partials)


# speedup vs baseline: 1.2169x; 1.0009x over previous
import jax
import jax.numpy as jnp
from jax import lax
from jax.experimental import pallas as pl
from jax.experimental.pallas import tpu as pltpu
from jax.experimental.pallas import tpu_sc as plsc

N = 10000
D = 128
E = 320000
NC = 2
NS = 16
NW = NC * NS
CHUNK = 128
K = 80
E_PAD = NW * K * CHUNK
ROWS_PER_TILE = 624
ZCHUNK = 104
NZ = 6
TAIL_OFF = NS * ROWS_PER_TILE
TAIL = N - TAIL_OFF


NCHUNKS = E_PAD // CHUNK
CB = NCHUNKS // 10


def _mm_body(x_ref, w_ref, ei_ref, ev_ref,
             sup_ref, src_ref, dst_ref, evp_ref):
    sup_ref[...] = jnp.dot(x_ref[...], w_ref[...],
                           preferred_element_type=jnp.float32)
    i = pl.program_id(0)
    r = lax.broadcasted_iota(jnp.int32, (CB, CHUNK), 0)
    cl = lax.broadcasted_iota(jnp.int32, (CB, CHUNK), 1)
    pos = (i * CB + r) * CHUNK + cl
    valid = pos < E
    spread = (pos * 13) % N
    src_ref[...] = jnp.where(valid, ei_ref[1].reshape(CB, CHUNK), spread)
    dst_ref[...] = jnp.where(valid, ei_ref[0].reshape(CB, CHUNK), spread)
    evp_ref[...] = jnp.where(valid, ev_ref[...].reshape(CB, CHUNK), 0.0)


def _matmul_prep(x, W, edge_index, edge_values):
    return pl.pallas_call(
        _mm_body,
        grid=(10,),
        in_specs=[
            pl.BlockSpec((N // 10, D), lambda i: (i, 0)),
            pl.BlockSpec((D, D), lambda i: (0, 0)),
            pl.BlockSpec((2, CB * CHUNK), lambda i: (0, i)),
            pl.BlockSpec((CB * CHUNK,), lambda i: (i,)),
        ],
        out_specs=[
            pl.BlockSpec((N // 10, D), lambda i: (i, 0)),
            pl.BlockSpec((CB, CHUNK), lambda i: (i, 0)),
            pl.BlockSpec((CB, CHUNK), lambda i: (i, 0)),
            pl.BlockSpec((CB, CHUNK), lambda i: (i, 0)),
        ],
        out_shape=[
            jax.ShapeDtypeStruct((N, D), jnp.float32),
            jax.ShapeDtypeStruct((NCHUNKS, CHUNK), jnp.int32),
            jax.ShapeDtypeStruct((NCHUNKS, CHUNK), jnp.int32),
            jax.ShapeDtypeStruct((NCHUNKS, CHUNK), jnp.float32),
        ],
    )(x, W, edge_index, edge_values)


def _sum_body(p_ref, o_ref):
    o_ref[...] = p_ref[0] + p_ref[1]


def _sum_partials(partials):
    return pl.pallas_call(
        _sum_body,
        grid=(10,),
        in_specs=[pl.BlockSpec((NC, N // 10, D), lambda i: (0, i, 0))],
        out_specs=pl.BlockSpec((N // 10, D), lambda i: (i, 0)),
        out_shape=jax.ShapeDtypeStruct((N, D), jnp.float32),
    )(partials)


def _spmm_body(support_hbm, src_hbm, dst_hbm, ev_hbm, out_hbm,
               src_v, dst_v, ev_v, rows_v, acc,
               gsem0, gsem1, msem0, msem1):
    c = lax.axis_index("c")
    s = lax.axis_index("s")
    wid = s * NC + c
    rows0, rows1 = rows_v.at[0], rows_v.at[1]

    def _zrow(i, carry):
        for g in range(8):
            rows_v[0, i, pl.ds(g * 16, 16)] = jnp.zeros((16,), jnp.float32)
        return carry
    with jax.named_scope("sc_zero"):
        lax.fori_loop(0, CHUNK, _zrow, 0)

    with jax.named_scope("sc_init"):
        pltpu.sync_copy(src_hbm.at[pl.ds(wid * K, K)], src_v)
        for kz in range(NZ):
            off = s * ROWS_PER_TILE + kz * ZCHUNK
            pltpu.sync_copy(rows0.at[pl.ds(0, ZCHUNK)],
                            acc.at[pl.ds(off, ZCHUNK)])

        @pl.when(s == 0)
        def _zero_tail():
            pltpu.sync_copy(rows0.at[pl.ds(0, TAIL)],
                            acc.at[pl.ds(TAIL_OFF, TAIL)])
        plsc.subcore_barrier()

    def _start(j, b, rows_b, gsem, msem):
        pltpu.async_copy(support_hbm.at[src_v.at[j]], rows_b, gsem)
        pltpu.async_copy(dst_hbm.at[wid * K + j], dst_v.at[b], msem)
        pltpu.async_copy(ev_hbm.at[wid * K + j], ev_v.at[b], msem)

    def _wait(j, b, rows_b, gsem, msem):
        pltpu.make_async_copy(support_hbm.at[src_v.at[j]], rows_b, gsem).wait()
        pltpu.make_async_copy(dst_hbm.at[wid * K + j], dst_v.at[b], msem).wait()
        pltpu.make_async_copy(ev_hbm.at[wid * K + j], ev_v.at[b], msem).wait()

    def _scale_scatter(b, rows_b):
        ev_b = ev_v.at[b]

        @plsc.parallel_loop(0, CHUNK // 16, unroll=2)
        def _scale(g16):
            vals16 = ev_b[pl.ds(g16 * 16, 16)]
            for l in range(16):
                e = g16 * 16 + l
                bc = jnp.broadcast_to(vals16[l], (16,))
                for g in range(8):
                    sl = pl.ds(g * 16, 16)
                    rows_b[e, sl] = rows_b[e, sl] * bc
        pltpu.sync_copy(rows_b, acc.at[dst_v.at[b]], add=True)

    with jax.named_scope("sc_prime"):
        _start(0, 0, rows0, gsem0, msem0)

    def _pair(i, carry):
        jj = 2 * i
        _start(jj + 1, 1, rows1, gsem1, msem1)
        _wait(jj, 0, rows0, gsem0, msem0)
        _scale_scatter(0, rows0)

        @pl.when(i + 1 < K // 2)
        def _prefetch_next():
            _start(jj + 2, 0, rows0, gsem0, msem0)
        _wait(jj + 1, 1, rows1, gsem1, msem1)
        _scale_scatter(1, rows1)
        return carry
    with jax.named_scope("sc_main"):
        lax.fori_loop(0, K // 2, _pair, 0)

    with jax.named_scope("sc_bar2"):
        plsc.subcore_barrier()
    with jax.named_scope("sc_wb"):
        for kz in range(NZ):
            off = s * ROWS_PER_TILE + kz * ZCHUNK
            pltpu.sync_copy(acc.at[pl.ds(off, ZCHUNK)],
                            out_hbm.at[c, pl.ds(off, ZCHUNK)])

    @pl.when(s == 0)
    def _write_tail():
        pltpu.sync_copy(acc.at[pl.ds(TAIL_OFF, TAIL)],
                        out_hbm.at[c, pl.ds(TAIL_OFF, TAIL)])


_spmm = pl.kernel(
    _spmm_body,
    out_type=jax.ShapeDtypeStruct((NC, N, D), jnp.float32),
    mesh=plsc.VectorSubcoreMesh(core_axis_name="c", subcore_axis_name="s"),
    scratch_types=[
        pltpu.VMEM((K, CHUNK), jnp.int32),
        pltpu.VMEM((2, CHUNK), jnp.int32),
        pltpu.VMEM((2, CHUNK), jnp.float32),
        pltpu.VMEM((2, CHUNK, D), jnp.float32),
        pltpu.VMEM_SHARED((N, D), jnp.float32),
        pltpu.SemaphoreType.DMA,
        pltpu.SemaphoreType.DMA,
        pltpu.SemaphoreType.DMA,
        pltpu.SemaphoreType.DMA,
    ],
)


@jax.jit
def kernel(x, edge_index, edge_values, W):
    support, src_p, dst_p, ev_p = _matmul_prep(x, W, edge_index, edge_values)
    partials = _spmm(support, src_p, dst_p, ev_p)
    return _sum_partials(partials)
